# baseline (speedup 1.0000x reference)
import jax
import jax.numpy as jnp
from jax.experimental import pallas as pl
from jax.experimental.pallas import tpu as pltpu

_TM = 200
_QD = 5
_TQ = _TM // _QD
_SLOTS = 3


def _fts_kernel(s1_ref, s2_ref, w_ref, out_ref):
    nh = w_ref.shape[1]
    w = w_ref[...]
    f1 = jnp.dot(s1_ref[...], w, preferred_element_type=jnp.float32)
    f2 = jnp.dot(s2_ref[...], w, preferred_element_type=jnp.float32)
    out_ref[:, :nh] = f1.astype(jnp.bfloat16)
    out_ref[:, nh:] = f2.astype(jnp.bfloat16)


def _gcn_kernel(adj_ref, fts_ref, wb_ref, mskc_ref, bias2_ref, a_ref,
                b_ref, sb1_ref, sb2_ref,
                o1_ref, o2_ref,
                buf_ref, sem_ref, g_ref, csum_ref):
    n = fts_ref.shape[0]
    nh = wb_ref.shape[0]
    steps = n // _TM
    fts = fts_ref[...]
    bias2 = bias2_ref[...]
    wb = wb_ref[...]
    a = a_ref[0, 0]

    def issue(t, slot):
        for k in range(_QD):
            pltpu.make_async_copy(
                adj_ref.at[0, pl.ds(t * _TM + k * _TQ, _TQ), :],
                buf_ref.at[slot, pl.ds(k * _TQ, _TQ), :],
                sem_ref.at[slot, k],
            ).start()

    csum_ref[...] = jnp.zeros_like(csum_ref)
    for d in range(_SLOTS):
        issue(d, d)

    def body(t, carry):
        slot = jax.lax.rem(t, _SLOTS)
        for k in range(_QD):
            pltpu.make_async_copy(
                adj_ref.at[0, pl.ds(t * _TM + k * _TQ, _TQ), :],
                buf_ref.at[slot, pl.ds(k * _TQ, _TQ), :],
                sem_ref.at[slot, k],
            ).wait()
        ab = buf_ref[slot].astype(jnp.bfloat16)
        acc = jnp.dot(ab, fts,
                      preferred_element_type=jnp.float32)
        out = acc + bias2
        h = jnp.where(out >= 0, out, a * out)
        g1 = jnp.dot(h[:, :nh].astype(jnp.bfloat16), wb,
                     preferred_element_type=jnp.float32)
        g2 = jnp.dot(h[:, nh:].astype(jnp.bfloat16), wb,
                     preferred_element_type=jnp.float32)
        sl = pl.ds(t * _TM, _TM)
        g_ref[sl, :nh] = g1.astype(jnp.bfloat16)
        g_ref[sl, nh:] = g2.astype(jnp.bfloat16)
        mk = mskc_ref[sl, :]
        csum_ref[...] += jnp.sum(h[:, :nh] * mk, axis=0, keepdims=True)

        @pl.when(t + _SLOTS < steps)
        def _():
            issue(t + _SLOTS, slot)

        return carry

    jax.lax.fori_loop(0, steps, body, 0)

    smsk = jnp.sum(mskc_ref[...])
    c = jax.nn.sigmoid(csum_ref[...] / smsk)
    b = b_ref[0, 0]
    chunk = 1000
    for j in range(n // chunk):
        sl = pl.ds(j * chunk, chunk)
        gj = g_ref[sl, :].astype(jnp.float32)
        o1_ref[sl, :] = (jnp.sum(gj[:, :nh] * c, axis=1, keepdims=True)
                         + b + sb1_ref[sl, :])
        o2_ref[sl, :] = (jnp.sum(gj[:, nh:] * c, axis=1, keepdims=True)
                         + b + sb2_ref[sl, :])


def kernel(seq1, seq2, adj, msk, samp_bias1, samp_bias2,
           W_fc, gcn_bias, prelu_a, bilin_W, bilin_b):
    n = adj.shape[1]
    nh = W_fc.shape[1]
    nin = W_fc.shape[0]

    tb = 2000

    s1 = seq1[0]
    s2 = seq2[0]

    fts = pl.pallas_call(
        _fts_kernel,
        grid=(n // tb,),
        in_specs=[
            pl.BlockSpec((tb, nin), lambda i: (i, 0)),
            pl.BlockSpec((tb, nin), lambda i: (i, 0)),
            pl.BlockSpec((nin, nh), lambda i: (0, 0)),
        ],
        out_specs=pl.BlockSpec((tb, 2 * nh), lambda i: (i, 0)),
        out_shape=jax.ShapeDtypeStruct((n, 2 * nh), jnp.bfloat16),
    )(s1, s2, W_fc)

    bias2 = jnp.concatenate([gcn_bias, gcn_bias]).reshape(1, 2 * nh)
    a2 = prelu_a.reshape(1, 1)
    b2 = bilin_b.reshape(1, 1)
    mskc = msk.reshape(n, 1)
    sb1 = samp_bias1.reshape(n, 1)
    sb2 = samp_bias2.reshape(n, 1)
    wb = bilin_W.astype(jnp.bfloat16)

    full = lambda r, c: pl.BlockSpec((r, c), lambda: (0, 0))
    o1, o2 = pl.pallas_call(
        _gcn_kernel,
        grid=(),
        in_specs=[
            pl.BlockSpec(memory_space=pltpu.HBM),
            full(n, 2 * nh),
            full(nh, nh),
            full(n, 1),
            full(1, 2 * nh),
            full(1, 1),
            full(1, 1),
            full(n, 1),
            full(n, 1),
        ],
        out_specs=[full(n, 1), full(n, 1)],
        out_shape=[
            jax.ShapeDtypeStruct((n, 1), jnp.float32),
            jax.ShapeDtypeStruct((n, 1), jnp.float32),
        ],
        scratch_shapes=[
            pltpu.VMEM((_SLOTS, _TM, n), jnp.float32),
            pltpu.SemaphoreType.DMA((_SLOTS, _QD)),
            pltpu.VMEM((n, 2 * nh), jnp.bfloat16),
            pltpu.VMEM((1, nh), jnp.float32),
        ],
        compiler_params=pltpu.CompilerParams(
            vmem_limit_bytes=63 * 1024 * 1024,
        ),
    )(adj, fts, wb, mskc, bias2, a2, b2, sb1, sb2)

    logits = jnp.concatenate([o1[:, 0], o2[:, 0]])[None, :]
    return logits


# speedup vs baseline: 1.0065x; 1.0065x over previous
import jax
import jax.numpy as jnp
from jax.experimental import pallas as pl
from jax.experimental.pallas import tpu as pltpu

_TM = 200
_QD = 5
_TQ = _TM // _QD
_SLOTS = 2


def _fts_kernel(s1_ref, s2_ref, w_ref, out_ref):
    nh = w_ref.shape[1]
    w = w_ref[...]
    f1 = jnp.dot(s1_ref[...], w, preferred_element_type=jnp.float32)
    f2 = jnp.dot(s2_ref[...], w, preferred_element_type=jnp.float32)
    out_ref[:, :nh] = f1.astype(jnp.bfloat16)
    out_ref[:, nh:] = f2.astype(jnp.bfloat16)


def _gcn_kernel(adj_ref, fts_ref, wb_ref, mskc_ref, bias2_ref, a_ref,
                b_ref, sb1_ref, sb2_ref,
                o1_ref, o2_ref,
                buf_ref, sem_ref, g_ref, csum_ref):
    n = fts_ref.shape[0]
    nh = wb_ref.shape[0]
    steps = n // _TM
    fts = fts_ref[...]
    bias2 = bias2_ref[...]
    wb = wb_ref[...]
    a = a_ref[0, 0]

    def issue(t, slot):
        for k in range(_QD):
            pltpu.make_async_copy(
                adj_ref.at[0, pl.ds(t * _TM + k * _TQ, _TQ), :],
                buf_ref.at[slot, pl.ds(k * _TQ, _TQ), :],
                sem_ref.at[slot, k],
            ).start()

    csum_ref[...] = jnp.zeros_like(csum_ref)
    for d in range(_SLOTS):
        issue(d, d)

    def body(t, carry):
        slot = jax.lax.rem(t, _SLOTS)
        for k in range(_QD):
            pltpu.make_async_copy(
                adj_ref.at[0, pl.ds(t * _TM + k * _TQ, _TQ), :],
                buf_ref.at[slot, pl.ds(k * _TQ, _TQ), :],
                sem_ref.at[slot, k],
            ).wait()
        ab = buf_ref[slot].astype(jnp.bfloat16)
        acc = jnp.dot(ab, fts,
                      preferred_element_type=jnp.float32)
        out = acc + bias2
        h = jnp.where(out >= 0, out, a * out)
        g1 = jnp.dot(h[:, :nh].astype(jnp.bfloat16), wb,
                     preferred_element_type=jnp.float32)
        g2 = jnp.dot(h[:, nh:].astype(jnp.bfloat16), wb,
                     preferred_element_type=jnp.float32)
        sl = pl.ds(t * _TM, _TM)
        g_ref[sl, :nh] = g1.astype(jnp.bfloat16)
        g_ref[sl, nh:] = g2.astype(jnp.bfloat16)
        mk = mskc_ref[sl, :]
        csum_ref[...] += jnp.sum(h[:, :nh] * mk, axis=0, keepdims=True)

        @pl.when(t + _SLOTS < steps)
        def _():
            issue(t + _SLOTS, slot)

        return carry

    jax.lax.fori_loop(0, steps, body, 0)

    smsk = jnp.sum(mskc_ref[...])
    c = jax.nn.sigmoid(csum_ref[...] / smsk)
    b = b_ref[0, 0]
    chunk = 1000
    for j in range(n // chunk):
        sl = pl.ds(j * chunk, chunk)
        gj = g_ref[sl, :].astype(jnp.float32)
        o1_ref[sl, :] = (jnp.sum(gj[:, :nh] * c, axis=1, keepdims=True)
                         + b + sb1_ref[sl, :])
        o2_ref[sl, :] = (jnp.sum(gj[:, nh:] * c, axis=1, keepdims=True)
                         + b + sb2_ref[sl, :])


def kernel(seq1, seq2, adj, msk, samp_bias1, samp_bias2,
           W_fc, gcn_bias, prelu_a, bilin_W, bilin_b):
    n = adj.shape[1]
    nh = W_fc.shape[1]
    nin = W_fc.shape[0]

    tb = 2000

    s1 = seq1[0]
    s2 = seq2[0]

    fts = pl.pallas_call(
        _fts_kernel,
        grid=(n // tb,),
        in_specs=[
            pl.BlockSpec((tb, nin), lambda i: (i, 0)),
            pl.BlockSpec((tb, nin), lambda i: (i, 0)),
            pl.BlockSpec((nin, nh), lambda i: (0, 0)),
        ],
        out_specs=pl.BlockSpec((tb, 2 * nh), lambda i: (i, 0)),
        out_shape=jax.ShapeDtypeStruct((n, 2 * nh), jnp.bfloat16),
    )(s1, s2, W_fc)

    bias2 = jnp.concatenate([gcn_bias, gcn_bias]).reshape(1, 2 * nh)
    a2 = prelu_a.reshape(1, 1)
    b2 = bilin_b.reshape(1, 1)
    mskc = msk.reshape(n, 1)
    sb1 = samp_bias1.reshape(n, 1)
    sb2 = samp_bias2.reshape(n, 1)
    wb = bilin_W.astype(jnp.bfloat16)

    full = lambda r, c: pl.BlockSpec((r, c), lambda: (0, 0))
    o1, o2 = pl.pallas_call(
        _gcn_kernel,
        grid=(),
        in_specs=[
            pl.BlockSpec(memory_space=pltpu.HBM),
            full(n, 2 * nh),
            full(nh, nh),
            full(n, 1),
            full(1, 2 * nh),
            full(1, 1),
            full(1, 1),
            full(n, 1),
            full(n, 1),
        ],
        out_specs=[full(n, 1), full(n, 1)],
        out_shape=[
            jax.ShapeDtypeStruct((n, 1), jnp.float32),
            jax.ShapeDtypeStruct((n, 1), jnp.float32),
        ],
        scratch_shapes=[
            pltpu.VMEM((_SLOTS, _TM, n), jnp.float32),
            pltpu.SemaphoreType.DMA((_SLOTS, _QD)),
            pltpu.VMEM((n, 2 * nh), jnp.bfloat16),
            pltpu.VMEM((1, nh), jnp.float32),
        ],
        compiler_params=pltpu.CompilerParams(
            vmem_limit_bytes=60 * 1024 * 1024,
        ),
    )(adj, fts, wb, mskc, bias2, a2, b2, sb1, sb2)

    logits = jnp.concatenate([o1[:, 0], o2[:, 0]])[None, :]
    return logits
